# baseline (speedup 1.0000x reference)
import functools

import jax
import jax.numpy as jnp
from jax import lax
from jax.experimental import pallas as pl
from jax.experimental.pallas import tpu as pltpu
from jax.experimental.pallas import tpu_sc as plsc

EMB = 16


_DW = 16384
_DS = _DW // 8
_DSH = 11


def _detile16(at):
    R, C = at.shape
    G = 128 // R
    grid = (C + _DW - 1) // _DW

    def body(a_ref, o_ref):
        ri = lax.broadcasted_iota(jnp.int32, (128, 128), 0)
        ci = lax.broadcasted_iota(jnp.int32, (128, 128), 1)
        eye = (ri == ci).astype(jnp.float32)

        def compute(a):
            ap = jnp.concatenate(
                [a[:, g * _DS:(g + 1) * _DS] for g in range(G)], axis=0)
            return jax.lax.dot_general(
                ap, eye, (((0,), (0,)), ((), ())),
                preferred_element_type=jnp.float32)

        pid = pl.program_id(0)

        @pl.when(pid != grid - 1)
        def _full():
            o_ref[...] = compute(a_ref[...])

        @pl.when(pid == grid - 1)
        def _tail():
            cols = lax.broadcasted_iota(jnp.int32, (R, _DW), 1) + pid * _DW
            a = jnp.where(cols < C, a_ref[...], 0.0)
            o_ref[...] = compute(a)

    return pl.pallas_call(
        body, grid=(grid,),
        in_specs=[pl.BlockSpec((R, _DW), lambda i: (0, i))],
        out_specs=pl.BlockSpec((_DS, 128), lambda i: (i, 0)),
        out_shape=jax.ShapeDtypeStruct((grid * _DS, 128), at.dtype),
    )(at)


def _pooled_sum_sc(x1d, emb, B, L):
    V, E = emb.shape
    assert E == EMB

    info = plsc.get_sparse_core_info()
    NC, NS = info.num_cores, info.num_subcores
    NW = NC * NS
    rows_per_w = B // NW
    CB = 8
    n_chunks = rows_per_w // CB
    gather_n = CB * L
    H0, H1 = 104, 96
    assert H0 + H1 == L and H0 % 8 == 0 and L % 8 == 0

    mesh = plsc.VectorSubcoreMesh(core_axis_name="c", subcore_axis_name="s")

    UNROLL = 40
    n_red = L // UNROLL

    @functools.partial(
        pl.kernel,
        out_type=jax.ShapeDtypeStruct((B, EMB), jnp.float32),
        mesh=mesh,
        scratch_types=[
            pltpu.VMEM((gather_n,), jnp.int32),
            pltpu.VMEM((gather_n,), jnp.int32),
            pltpu.VMEM((gather_n,), jnp.int32),
            pltpu.VMEM((gather_n,), jnp.int32),
            pltpu.VMEM((gather_n, EMB), jnp.float32),
            pltpu.VMEM((gather_n, EMB), jnp.float32),
            pltpu.VMEM((CB, EMB), jnp.float32),
            pltpu.VMEM((CB, EMB), jnp.float32),
            pltpu.SemaphoreType.DMA,
            pltpu.SemaphoreType.DMA,
            pltpu.SemaphoreType.DMA,
            pltpu.SemaphoreType.DMA,
            pltpu.SemaphoreType.DMA,
            pltpu.SemaphoreType.DMA,
            pltpu.SemaphoreType.DMA,
        ],
        compiler_params=pltpu.CompilerParams(use_tc_tiling_on_sc=False),
    )
    def sc_kernel(x_hbm, emb_hbm, out_hbm,
                  idx_0, idx_1, idx_2, idx_3, rows_a, rows_b,
                  pooled_a, pooled_b,
                  sem_a, sem_b, isem_0, isem_1, isem_2, isem_3, osem):
        wid = lax.axis_index("s") * NC + lax.axis_index("c")
        row0 = wid * rows_per_w
        idxs = (idx_0, idx_1, idx_2, idx_3)
        isems = (isem_0, isem_1, isem_2, isem_3)

        def idx_desc(k, slot):
            return pltpu.make_async_copy(
                x_hbm.at[pl.ds((row0 + k * CB) * L, gather_n)],
                idxs[slot], isems[slot])

        def remap(idx_ref):
            def rbody(i, _):
                sl = pl.ds(i * EMB, EMB)
                kv = idx_ref[sl]
                e = jnp.bitwise_and(kv, _DW - 1)
                idx_ref[sl] = (kv - e) | ((e & (_DS - 1)) << 3) | (e >> _DSH)
                return 0
            lax.fori_loop(0, gather_n // EMB, rbody, 0, unroll=4)

        def gather_descs(idx_ref, rows_ref, sem):
            descs = []
            for r in range(CB):
                for off, n in ((0, H0), (H0, H1)):
                    descs.append(pltpu.make_async_copy(
                        emb_hbm.at[idx_ref.at[pl.ds(r * L + off, n)]],
                        rows_ref.at[pl.ds(r * L + off, n)],
                        sem,
                    ))
            return descs

        def out_desc(k, pooled_ref):
            return pltpu.make_async_copy(
                pooled_ref, out_hbm.at[pl.ds(row0 + k * CB, CB)], osem)

        for s in range(4):
            idx_desc(s, s).start()
        idx_desc(0, 0).wait()
        remap(idx_0)
        for d in gather_descs(idx_0, rows_a, sem_a):
            d.start()

        rbufs = ((rows_a, sem_a), (rows_b, sem_b))
        pbufs = (pooled_a, pooled_b)

        def quad_body(g, _):
            for b in range(4):
                k = 4 * g + b
                rows_c, sem_c = rbufs[b % 2]
                rows_n, sem_n = rbufs[(b + 1) % 2]
                pooled_v = pbufs[b % 2]

                for d in gather_descs(idxs[b], rows_c, sem_c):
                    d.wait()

                @pl.when(k + 4 < n_chunks)
                def _pf_idx():
                    idx_desc(k + 4, b).start()

                @pl.when(k + 1 < n_chunks)
                def _pf_gather():
                    nb = (b + 1) % 4
                    idx_desc(k + 1, nb).wait()
                    remap(idxs[nb])
                    for d in gather_descs(idxs[nb], rows_n, sem_n):
                        d.start()

                @pl.when(k >= 2)
                def _drain_out():
                    out_desc(k, pooled_v).wait()

                for r in range(CB):
                    base = r * L

                    def red_body(i, accs, base=base):
                        a0, a1, a2, a3 = accs
                        off = base + i * UNROLL
                        for j in range(UNROLL):
                            v = rows_c[off + j]
                            if j % 4 == 0:
                                a0 = a0 + v
                            elif j % 4 == 1:
                                a1 = a1 + v
                            elif j % 4 == 2:
                                a2 = a2 + v
                            else:
                                a3 = a3 + v
                        return (a0, a1, a2, a3)

                    z = jnp.zeros((EMB,), jnp.float32)
                    a0, a1, a2, a3 = lax.fori_loop(
                        0, n_red, red_body, (z, z, z, z))
                    pooled_v[r] = (a0 + a1) + (a2 + a3)

                out_desc(k, pooled_v).start()
            return 0

        lax.fori_loop(0, n_chunks // 4, quad_body, 0)
        out_desc(n_chunks - 2, pooled_a).wait()
        out_desc(n_chunks - 1, pooled_b).wait()

    return sc_kernel(x1d, emb)


def _mlp_tc(h2d, W1s, b1, W2, b2):
    Bd8 = h2d.shape[0]
    BLK = 1024
    eye8 = jnp.eye(8, dtype=jnp.float32)
    W1d = jnp.kron(eye8, W1s)
    b1d = jnp.tile(b1, 8).reshape(1, 128)
    W2d = jnp.kron(eye8, W2)
    b2d = jnp.tile(b2, 8).reshape(1, 16)

    def body(h_ref, w1_ref, b1_ref, w2_ref, b2_ref, o_ref):
        z = jnp.dot(h_ref[...], w1_ref[...],
                    preferred_element_type=jnp.float32) + b1_ref[...]
        z = jnp.maximum(z, 0.0)
        o_ref[...] = jnp.dot(z, w2_ref[...],
                             preferred_element_type=jnp.float32) + b2_ref[...]

    out = pl.pallas_call(
        body,
        grid=(Bd8 // BLK,),
        in_specs=[
            pl.BlockSpec((BLK, 128), lambda i: (i, 0)),
            pl.BlockSpec((128, 128), lambda i: (0, 0)),
            pl.BlockSpec((1, 128), lambda i: (0, 0)),
            pl.BlockSpec((128, EMB), lambda i: (0, 0)),
            pl.BlockSpec((1, EMB), lambda i: (0, 0)),
        ],
        out_specs=pl.BlockSpec((BLK, EMB), lambda i: (i, 0)),
        out_shape=jax.ShapeDtypeStruct((Bd8, EMB), jnp.float32),
    )(h2d, W1d, b1d, W2d, b2d)
    return out.reshape(Bd8 * 8, 2)


def kernel(x, emb, W1, b1, W2, b2):
    B, L = x.shape
    emb_lin = _detile16(emb.T)
    Vp = emb_lin.shape[0] * 128 // EMB
    pooled = _pooled_sum_sc(x.astype(jnp.int32).reshape(B * L),
                            emb_lin.reshape(Vp, EMB), B, L)
    return _mlp_tc(pooled.reshape(B // 8, 128),
                   W1 * (1.0 / L), b1, W2, b2)


# speedup vs baseline: 1.0030x; 1.0030x over previous
import functools

import jax
import jax.numpy as jnp
from jax import lax
from jax.experimental import pallas as pl
from jax.experimental.pallas import tpu as pltpu
from jax.experimental.pallas import tpu_sc as plsc

EMB = 16


def _remap_x(xt):
    R, C = xt.shape
    W = 2048

    def body(a_ref, o_ref):
        k = a_ref[...]
        e = jnp.bitwise_and(k, _DW - 1)
        o_ref[...] = (k - e) | ((e & (_DS - 1)) << 3) | (e >> _DSH)

    return pl.pallas_call(
        body, grid=(C // W,),
        in_specs=[pl.BlockSpec((R, W), lambda i: (0, i))],
        out_specs=pl.BlockSpec((R, W), lambda i: (0, i)),
        out_shape=jax.ShapeDtypeStruct((R, C), xt.dtype),
    )(xt)


_DW = 16384
_DS = _DW // 8
_DSH = 11


def _detile16(at):
    R, C = at.shape
    G = 128 // R
    grid = (C + _DW - 1) // _DW

    def body(a_ref, o_ref):
        ri = lax.broadcasted_iota(jnp.int32, (128, 128), 0)
        ci = lax.broadcasted_iota(jnp.int32, (128, 128), 1)
        eye = (ri == ci).astype(jnp.float32)

        def compute(a):
            ap = jnp.concatenate(
                [a[:, g * _DS:(g + 1) * _DS] for g in range(G)], axis=0)
            return jax.lax.dot_general(
                ap, eye, (((0,), (0,)), ((), ())),
                preferred_element_type=jnp.float32)

        pid = pl.program_id(0)

        @pl.when(pid != grid - 1)
        def _full():
            o_ref[...] = compute(a_ref[...])

        @pl.when(pid == grid - 1)
        def _tail():
            cols = lax.broadcasted_iota(jnp.int32, (R, _DW), 1) + pid * _DW
            a = jnp.where(cols < C, a_ref[...], 0.0)
            o_ref[...] = compute(a)

    return pl.pallas_call(
        body, grid=(grid,),
        in_specs=[pl.BlockSpec((R, _DW), lambda i: (0, i))],
        out_specs=pl.BlockSpec((_DS, 128), lambda i: (i, 0)),
        out_shape=jax.ShapeDtypeStruct((grid * _DS, 128), at.dtype),
    )(at)


def _pooled_sum_sc(x1d, emb, B, L):
    V, E = emb.shape
    assert E == EMB

    info = plsc.get_sparse_core_info()
    NC, NS = info.num_cores, info.num_subcores
    NW = NC * NS
    rows_per_w = B // NW
    CB = 8
    n_chunks = rows_per_w // CB
    gather_n = CB * L
    H0, H1 = 104, 96
    assert H0 + H1 == L and H0 % 8 == 0 and L % 8 == 0

    mesh = plsc.VectorSubcoreMesh(core_axis_name="c", subcore_axis_name="s")

    UNROLL = 40
    n_red = L // UNROLL

    @functools.partial(
        pl.kernel,
        out_type=jax.ShapeDtypeStruct((B, EMB), jnp.float32),
        mesh=mesh,
        scratch_types=[
            pltpu.VMEM((gather_n,), jnp.int32),
            pltpu.VMEM((gather_n,), jnp.int32),
            pltpu.VMEM((gather_n,), jnp.int32),
            pltpu.VMEM((gather_n,), jnp.int32),
            pltpu.VMEM((gather_n, EMB), jnp.float32),
            pltpu.VMEM((gather_n, EMB), jnp.float32),
            pltpu.VMEM((CB, EMB), jnp.float32),
            pltpu.VMEM((CB, EMB), jnp.float32),
            pltpu.SemaphoreType.DMA,
            pltpu.SemaphoreType.DMA,
            pltpu.SemaphoreType.DMA,
            pltpu.SemaphoreType.DMA,
            pltpu.SemaphoreType.DMA,
            pltpu.SemaphoreType.DMA,
            pltpu.SemaphoreType.DMA,
        ],
        compiler_params=pltpu.CompilerParams(use_tc_tiling_on_sc=False),
    )
    def sc_kernel(x_hbm, emb_hbm, out_hbm,
                  idx_0, idx_1, idx_2, idx_3, rows_a, rows_b,
                  pooled_a, pooled_b,
                  sem_a, sem_b, isem_0, isem_1, isem_2, isem_3, osem):
        wid = lax.axis_index("s") * NC + lax.axis_index("c")
        row0 = wid * rows_per_w
        idxs = (idx_0, idx_1, idx_2, idx_3)
        isems = (isem_0, isem_1, isem_2, isem_3)

        def idx_desc(k, slot):
            return pltpu.make_async_copy(
                x_hbm.at[pl.ds((row0 + k * CB) * L, gather_n)],
                idxs[slot], isems[slot])

        def gather_descs(idx_ref, rows_ref, sem):
            descs = []
            for r in range(CB):
                for off, n in ((0, H0), (H0, H1)):
                    descs.append(pltpu.make_async_copy(
                        emb_hbm.at[idx_ref.at[pl.ds(r * L + off, n)]],
                        rows_ref.at[pl.ds(r * L + off, n)],
                        sem,
                    ))
            return descs

        def out_desc(k, pooled_ref):
            return pltpu.make_async_copy(
                pooled_ref, out_hbm.at[pl.ds(row0 + k * CB, CB)], osem)

        for s in range(4):
            idx_desc(s, s).start()
        idx_desc(0, 0).wait()
        for d in gather_descs(idx_0, rows_a, sem_a):
            d.start()

        rbufs = ((rows_a, sem_a), (rows_b, sem_b))
        pbufs = (pooled_a, pooled_b)

        def quad_body(g, _):
            for b in range(4):
                k = 4 * g + b
                rows_c, sem_c = rbufs[b % 2]
                rows_n, sem_n = rbufs[(b + 1) % 2]
                pooled_v = pbufs[b % 2]

                for d in gather_descs(idxs[b], rows_c, sem_c):
                    d.wait()

                @pl.when(k + 4 < n_chunks)
                def _pf_idx():
                    idx_desc(k + 4, b).start()

                @pl.when(k + 1 < n_chunks)
                def _pf_gather():
                    nb = (b + 1) % 4
                    idx_desc(k + 1, nb).wait()
                    for d in gather_descs(idxs[nb], rows_n, sem_n):
                        d.start()

                @pl.when(k >= 2)
                def _drain_out():
                    out_desc(k, pooled_v).wait()

                for r in range(CB):
                    base = r * L

                    def red_body(i, accs, base=base):
                        a0, a1, a2, a3 = accs
                        off = base + i * UNROLL
                        for j in range(UNROLL):
                            v = rows_c[off + j]
                            if j % 4 == 0:
                                a0 = a0 + v
                            elif j % 4 == 1:
                                a1 = a1 + v
                            elif j % 4 == 2:
                                a2 = a2 + v
                            else:
                                a3 = a3 + v
                        return (a0, a1, a2, a3)

                    z = jnp.zeros((EMB,), jnp.float32)
                    a0, a1, a2, a3 = lax.fori_loop(
                        0, n_red, red_body, (z, z, z, z))
                    pooled_v[r] = (a0 + a1) + (a2 + a3)

                out_desc(k, pooled_v).start()
            return 0

        lax.fori_loop(0, n_chunks // 4, quad_body, 0)
        out_desc(n_chunks - 2, pooled_a).wait()
        out_desc(n_chunks - 1, pooled_b).wait()

    return sc_kernel(x1d, emb)


def _mlp_tc(h2d, W1s, b1, W2, b2):
    Bd8 = h2d.shape[0]
    BLK = 1024
    eye8 = jnp.eye(8, dtype=jnp.float32)
    W1d = jnp.kron(eye8, W1s)
    b1d = jnp.tile(b1, 8).reshape(1, 128)
    W2d = jnp.kron(eye8, W2)
    b2d = jnp.tile(b2, 8).reshape(1, 16)

    def body(h_ref, w1_ref, b1_ref, w2_ref, b2_ref, o_ref):
        z = jnp.dot(h_ref[...], w1_ref[...],
                    preferred_element_type=jnp.float32) + b1_ref[...]
        z = jnp.maximum(z, 0.0)
        o_ref[...] = jnp.dot(z, w2_ref[...],
                             preferred_element_type=jnp.float32) + b2_ref[...]

    out = pl.pallas_call(
        body,
        grid=(Bd8 // BLK,),
        in_specs=[
            pl.BlockSpec((BLK, 128), lambda i: (i, 0)),
            pl.BlockSpec((128, 128), lambda i: (0, 0)),
            pl.BlockSpec((1, 128), lambda i: (0, 0)),
            pl.BlockSpec((128, EMB), lambda i: (0, 0)),
            pl.BlockSpec((1, EMB), lambda i: (0, 0)),
        ],
        out_specs=pl.BlockSpec((BLK, EMB), lambda i: (i, 0)),
        out_shape=jax.ShapeDtypeStruct((Bd8, EMB), jnp.float32),
    )(h2d, W1d, b1d, W2d, b2d)
    return out.reshape(Bd8 * 8, 2)


def kernel(x, emb, W1, b1, W2, b2):
    B, L = x.shape
    emb_lin = _detile16(emb.T)
    Vp = emb_lin.shape[0] * 128 // EMB
    x_remap = _remap_x(x.T.astype(jnp.int32)).T
    pooled = _pooled_sum_sc(x_remap.reshape(B * L),
                            emb_lin.reshape(Vp, EMB), B, L)
    return _mlp_tc(pooled.reshape(B // 8, 128),
                   W1 * (1.0 / L), b1, W2, b2)
